# baseline (speedup 1.0000x reference)
import functools

import jax
import jax.numpy as jnp
from jax import lax
from jax.experimental import pallas as pl
from jax.experimental.pallas import tpu as pltpu
from jax.experimental.pallas import tpu_sc as plsc

NUM_CORES = 2
NUM_SUBCORES = 16
NUM_WORKERS = NUM_CORES * NUM_SUBCORES
HALVES = 2
DEPTH = 5


def _sc_gather(idx3, table, b, p, per_w, d):
    w_per_b = p // per_w
    mesh = plsc.VectorSubcoreMesh(
        core_axis_name="c", subcore_axis_name="s",
        num_cores=NUM_CORES, num_subcores=NUM_SUBCORES,
    )

    @functools.partial(
        pl.kernel,
        out_type=jax.ShapeDtypeStruct((p, b, d), jnp.float32),
        mesh=mesh,
        scratch_types=[
            pltpu.VMEM((per_w, 8), jnp.int32),
        ]
        + [pltpu.VMEM((1, d // HALVES), jnp.float32)] * DEPTH
        + [pltpu.SemaphoreType.DMA] * (2 * DEPTH),
        compiler_params=pltpu.CompilerParams(
            disable_bounds_checks=True,
            disable_semaphore_checks=True,
        ),
    )
    def k(idx_hbm, table_hbm, out_hbm, idx_v, *bufs_sems):
        bufs = bufs_sems[:DEPTH]
        gsem = bufs_sems[DEPTH:2 * DEPTH]
        ssem = bufs_sems[2 * DEPTH:]
        hd = d // HALVES
        n_ch = per_w * HALVES
        wid = lax.axis_index("s") * NUM_CORES + lax.axis_index("c")
        b_idx = wid // w_per_b
        p0 = (wid % w_per_b) * per_w
        pltpu.sync_copy(idx_hbm.at[wid], idx_v)

        def gather(j):
            i, h = j // HALVES, j % HALVES
            return pltpu.async_copy(
                table_hbm.at[idx_v.at[i, pl.ds(0, 1)], pl.ds(h * hd, hd)],
                bufs[j % DEPTH], gsem[j % DEPTH])

        gathers = [None] * n_ch
        scatters = [None] * n_ch
        for j in range(DEPTH - 1):
            gathers[j] = gather(j)
        for j in range(n_ch):
            if j >= 1:
                scatters[j - 1].wait()
            if j + DEPTH - 1 < n_ch:
                gathers[j + DEPTH - 1] = gather(j + DEPTH - 1)
            gathers[j].wait()
            i, h = j // HALVES, j % HALVES
            scatters[j] = pltpu.async_copy(
                bufs[j % DEPTH],
                out_hbm.at[pl.ds(p0 + i, 1), b_idx, pl.ds(h * hd, hd)],
                ssem[j % DEPTH])
        scatters[n_ch - 1].wait()

    return k(idx3, table)


def kernel(prefix, table):
    b, p = prefix.shape
    v, d = table.shape
    n = b * p
    per_w = n // NUM_WORKERS
    assert n % NUM_WORKERS == 0 and p % per_w == 0
    idx3 = jnp.broadcast_to(
        prefix.reshape(NUM_WORKERS, per_w, 1).astype(jnp.int32),
        (NUM_WORKERS, per_w, 8),
    )
    out_t = _sc_gather(idx3, table, b, p, per_w, d)
    return jnp.transpose(out_t, (1, 0, 2))


# speedup vs baseline: 1.0157x; 1.0157x over previous
import functools

import jax
import jax.numpy as jnp
from jax import lax
from jax.experimental import pallas as pl
from jax.experimental.pallas import tpu as pltpu
from jax.experimental.pallas import tpu_sc as plsc

NUM_CORES = 2
NUM_SUBCORES = 16
NUM_WORKERS = NUM_CORES * NUM_SUBCORES
HALVES = 1
DEPTH = 2


def _sc_gather(idx3, table, b, p, per_w, d):
    w_per_b = p // per_w
    mesh = plsc.VectorSubcoreMesh(
        core_axis_name="c", subcore_axis_name="s",
        num_cores=NUM_CORES, num_subcores=NUM_SUBCORES,
    )

    @functools.partial(
        pl.kernel,
        out_type=jax.ShapeDtypeStruct((p, b, d), jnp.float32),
        mesh=mesh,
        scratch_types=[
            pltpu.VMEM((per_w, 8), jnp.int32),
        ]
        + [pltpu.VMEM((1, d // HALVES), jnp.float32)] * DEPTH
        + [pltpu.SemaphoreType.DMA] * (2 * DEPTH),
        compiler_params=pltpu.CompilerParams(
            disable_bounds_checks=True,
            disable_semaphore_checks=True,
        ),
    )
    def k(idx_hbm, table_hbm, out_hbm, idx_v, *bufs_sems):
        bufs = bufs_sems[:DEPTH]
        gsem = bufs_sems[DEPTH:2 * DEPTH]
        ssem = bufs_sems[2 * DEPTH:]
        hd = d // HALVES
        n_ch = per_w * HALVES
        wid = lax.axis_index("s") * NUM_CORES + lax.axis_index("c")
        b_idx = wid // w_per_b
        p0 = (wid % w_per_b) * per_w
        pltpu.sync_copy(idx_hbm.at[wid], idx_v)

        def gather(j):
            i, h = j // HALVES, j % HALVES
            return pltpu.async_copy(
                table_hbm.at[idx_v.at[i, pl.ds(0, 1)], pl.ds(h * hd, hd)],
                bufs[j % DEPTH], gsem[j % DEPTH])

        gathers = [None] * n_ch
        scatters = [None] * n_ch
        for j in range(DEPTH - 1):
            gathers[j] = gather(j)
        for j in range(n_ch):
            if j >= 1:
                scatters[j - 1].wait()
            if j + DEPTH - 1 < n_ch:
                gathers[j + DEPTH - 1] = gather(j + DEPTH - 1)
            gathers[j].wait()
            i, h = j // HALVES, j % HALVES
            scatters[j] = pltpu.async_copy(
                bufs[j % DEPTH],
                out_hbm.at[pl.ds(p0 + i, 1), b_idx, pl.ds(h * hd, hd)],
                ssem[j % DEPTH])
        scatters[n_ch - 1].wait()

    return k(idx3, table)


def kernel(prefix, table):
    b, p = prefix.shape
    v, d = table.shape
    n = b * p
    per_w = n // NUM_WORKERS
    assert n % NUM_WORKERS == 0 and p % per_w == 0
    idx3 = jnp.broadcast_to(
        prefix.reshape(NUM_WORKERS, per_w, 1).astype(jnp.int32),
        (NUM_WORKERS, per_w, 8),
    )
    out_t = _sc_gather(idx3, table, b, p, per_w, d)
    return jnp.transpose(out_t, (1, 0, 2))
